# baseline (speedup 1.0000x reference)
import functools

import jax
import jax.numpy as jnp
from jax import lax
from jax.experimental import pallas as pl
from jax.experimental.pallas import tpu as pltpu
from jax.experimental.pallas import tpu_sc as plsc

L = 16
FPAD = 112
NCHUNK = FPAD // L


def _build_sc_call(batch, fields, rows_per_w, num_workers):
    mesh = plsc.VectorSubcoreMesh(core_axis_name="c", subcore_axis_name="s")
    words = rows_per_w * fields

    @functools.partial(
        pl.kernel,
        mesh=mesh,
        out_type=jax.ShapeDtypeStruct((batch,), jnp.float32),
        compiler_params=pltpu.CompilerParams(needs_layout_passes=False),
        scratch_types=[
            pltpu.VMEM((words + L,), jnp.int32),
            pltpu.VMEM((FPAD,), jnp.float32),
            pltpu.VMEM((FPAD,), jnp.float32),
            pltpu.VMEM((FPAD,), jnp.float32),
            pltpu.VMEM((rows_per_w,), jnp.float32),
        ],
    )
    def sc_call(x_hbm, a_hbm, e_hbm, d_hbm, out_hbm, x_v, a_v, e_v, d_v, out_v):
        wid = lax.axis_index("s") * 2 + lax.axis_index("c")
        pltpu.sync_copy(a_hbm, a_v)
        pltpu.sync_copy(e_hbm, e_v)
        pltpu.sync_copy(d_hbm, d_v)
        pltpu.sync_copy(x_hbm.at[pl.ds(wid * words, words)], x_v.at[pl.ds(0, words)])
        x_v[pl.ds(words, L)] = jnp.zeros((L,), jnp.int32)

        av = [a_v[pl.ds(c * L, L)] for c in range(NCHUNK)]
        ev = [e_v[pl.ds(c * L, L)] for c in range(NCHUNK)]
        dv = [d_v[pl.ds(c * L, L)] for c in range(NCHUNK)]
        last_lane = lax.iota(jnp.int32, L) == (L - 1)

        def row_body(r, carry):
            base = r * fields
            acc = jnp.zeros((L,), jnp.float32)
            for c in range(NCHUNK):
                u = x_v[pl.ds(base + c * L, L)].astype(jnp.float32)
                acc = acc + (av[c] + u * (ev[c] + u * dv[c]))
            cs = plsc.cumsum(acc)
            plsc.store_scatter(
                out_v, [jnp.full((L,), r, jnp.int32)], cs, mask=last_lane
            )
            return carry

        lax.fori_loop(0, rows_per_w, row_body, 0)
        pltpu.sync_copy(out_v, out_hbm.at[pl.ds(wid * rows_per_w, rows_per_w)])

    return sc_call


def kernel(x, table, W, b):
    batch, fields = x.shape
    num_workers = 32
    rows_per_w = batch // num_workers

    w = W.reshape(-1).astype(jnp.float32)
    t0, t1, t2 = table[0], table[1], table[2]
    d = w * ((t2 - (t1 + t1)) + t0) * 0.5
    e = w * (t1 - t0) - d
    a = w * t0
    pad = jnp.zeros((FPAD - fields,), jnp.float32)
    A = jnp.concatenate([a, pad]).at[FPAD - 1].set(b[0])
    E = jnp.concatenate([e, pad])
    D = jnp.concatenate([d, pad])

    sc_call = _build_sc_call(batch, fields, rows_per_w, num_workers)
    return sc_call(x.reshape(-1), A, E, D).reshape(batch, 1)


# speedup vs baseline: 1.1295x; 1.1295x over previous
import functools

import jax
import jax.numpy as jnp
from jax import lax
from jax.experimental import pallas as pl
from jax.experimental.pallas import tpu as pltpu
from jax.experimental.pallas import tpu_sc as plsc

L = 16
NCHUNK = 7


def _build_sc_call(batch, fields, rows_per_w):
    mesh = plsc.VectorSubcoreMesh(core_axis_name="c", subcore_axis_name="s")
    tail_off = fields - L

    @functools.partial(
        pl.kernel,
        mesh=mesh,
        out_type=jax.ShapeDtypeStruct((batch,), jnp.float32),
        compiler_params=pltpu.CompilerParams(needs_layout_passes=False),
        scratch_types=[
            pltpu.VMEM((rows_per_w, fields), jnp.int32),
            pltpu.VMEM((NCHUNK * L,), jnp.float32),
            pltpu.VMEM((NCHUNK * L,), jnp.float32),
            pltpu.VMEM((NCHUNK * L,), jnp.float32),
            pltpu.VMEM((rows_per_w,), jnp.float32),
        ],
    )
    def sc_call(x_hbm, a_hbm, e_hbm, d_hbm, out_hbm, x_v, a_v, e_v, d_v, out_v):
        wid = lax.axis_index("s") * 2 + lax.axis_index("c")
        pltpu.sync_copy(a_hbm, a_v)
        pltpu.sync_copy(e_hbm, e_v)
        pltpu.sync_copy(d_hbm, d_v)
        pltpu.sync_copy(x_hbm.at[pl.ds(wid * rows_per_w, rows_per_w), :], x_v)

        av = [a_v[pl.ds(c * L, L)] for c in range(NCHUNK)]
        ev = [e_v[pl.ds(c * L, L)] for c in range(NCHUNK)]
        dv = [d_v[pl.ds(c * L, L)] for c in range(NCHUNK)]
        offs = [c * L for c in range(NCHUNK - 1)] + [tail_off]
        last_lane = lax.iota(jnp.int32, L) == (L - 1)

        def row_body(r, carry):
            acc = jnp.zeros((L,), jnp.float32)
            for c in range(NCHUNK):
                u = x_v[r, pl.ds(offs[c], L)].astype(jnp.float32)
                acc = acc + (av[c] + u * (ev[c] + u * dv[c]))
            cs = plsc.cumsum(acc)
            plsc.store_scatter(
                out_v, [jnp.full((L,), r, jnp.int32)], cs, mask=last_lane
            )
            return carry

        lax.fori_loop(0, rows_per_w, row_body, 0)
        pltpu.sync_copy(out_v, out_hbm.at[pl.ds(wid * rows_per_w, rows_per_w)])

    return sc_call


def kernel(x, table, W, b):
    batch, fields = x.shape
    rows_per_w = batch // 32

    w = W.reshape(-1).astype(jnp.float32)
    t0, t1, t2 = table[0], table[1], table[2]
    d = w * ((t2 - (t1 + t1)) + t0) * 0.5
    e = w * (t1 - t0) - d
    a = w * t0
    a = a.at[fields - 1].add(b[0])

    zpad = jnp.zeros((12,), jnp.float32)
    def chunkify(v):
        return jnp.concatenate([v[: 6 * L], zpad, v[6 * L : fields]])

    A, E, D = chunkify(a), chunkify(e), chunkify(d)

    sc_call = _build_sc_call(batch, fields, rows_per_w)
    return sc_call(x, A, E, D).reshape(batch, 1)


# speedup vs baseline: 1.4941x; 1.3228x over previous
import functools

import jax
import jax.numpy as jnp
from jax import lax
from jax.experimental import pallas as pl
from jax.experimental.pallas import tpu as pltpu
from jax.experimental.pallas import tpu_sc as plsc

L = 16


def _build_sc_call(batch, fields, cols_per_w):
    mesh = plsc.VectorSubcoreMesh(core_axis_name="c", subcore_axis_name="s")
    ngrp = cols_per_w // L

    @functools.partial(
        pl.kernel,
        mesh=mesh,
        out_type=jax.ShapeDtypeStruct((batch,), jnp.float32),
        compiler_params=pltpu.CompilerParams(needs_layout_passes=False),
        scratch_types=[
            pltpu.VMEM((fields, cols_per_w), jnp.int32),
            pltpu.VMEM((fields,), jnp.float32),
            pltpu.VMEM((fields,), jnp.float32),
            pltpu.VMEM((8,), jnp.float32),
            pltpu.VMEM((cols_per_w,), jnp.float32),
        ],
    )
    def sc_call(xt_hbm, e_hbm, d_hbm, s_hbm, out_hbm, xt_v, e_v, d_v, s_v, out_v):
        wid = lax.axis_index("s") * 2 + lax.axis_index("c")
        base = wid * cols_per_w
        pltpu.sync_copy(e_hbm, e_v)
        pltpu.sync_copy(d_hbm, d_v)
        pltpu.sync_copy(s_hbm, s_v)
        pltpu.sync_copy(xt_hbm.at[:, pl.ds(base, cols_per_w)], xt_v)

        init = plsc.load_gather(s_v, [jnp.zeros((L,), jnp.int32)])

        def field_body(f, accs):
            fidx = jnp.full((L,), f, jnp.int32)
            evec = plsc.load_gather(e_v, [fidx])
            dvec = plsc.load_gather(d_v, [fidx])
            out = []
            for j in range(ngrp):
                u = xt_v[f, pl.ds(j * L, L)].astype(jnp.float32)
                out.append(accs[j] + u * (evec + u * dvec))
            return tuple(out)

        accs = lax.fori_loop(0, fields, field_body, (init,) * ngrp)
        for j in range(ngrp):
            out_v[pl.ds(j * L, L)] = accs[j]
        pltpu.sync_copy(out_v, out_hbm.at[pl.ds(base, cols_per_w)])

    return sc_call


def kernel(x, table, W, b):
    batch, fields = x.shape
    cols_per_w = batch // 32

    w = W.reshape(-1).astype(jnp.float32)
    t0, t1, t2 = table[0], table[1], table[2]
    d = w * ((t2 - (t1 + t1)) + t0) * 0.5
    e = w * (t1 - t0) - d
    s = jnp.zeros((8,), jnp.float32).at[0].set(jnp.sum(w) * t0 + b[0])

    sc_call = _build_sc_call(batch, fields, cols_per_w)
    return sc_call(x.T, e, d, s).reshape(batch, 1)
